# baseline (speedup 1.0000x reference)
import functools

import jax
import jax.numpy as jnp
from jax.experimental import pallas as pl

B, D_IN, D_FEAT, N_CLS = 16384, 128, 64, 5
BLOCK_B = 2048


def _fused_kernel(x_ref, wf_ref, bf_ref, wfc_ref, bfc_ref, out_ref, pred_ref):
    feats = jnp.maximum(
        jnp.dot(x_ref[...], wf_ref[...], preferred_element_type=jnp.float32)
        + bf_ref[...],
        0.0,
    )
    out = (
        jnp.dot(feats, wfc_ref[...], preferred_element_type=jnp.float32)
        + bfc_ref[...]
    )
    out_ref[...] = out
    maxv = jnp.max(out, axis=-1, keepdims=True)
    idx = jax.lax.broadcasted_iota(jnp.int32, out.shape, 1)
    pred = jnp.min(jnp.where(out == maxv, idx, N_CLS), axis=-1, keepdims=True)
    pred_ref[...] = pred


@jax.jit
def _run(x, W_feat, b_feat, W_fc, b_fc):
    grid = (B // BLOCK_B,)
    out, pred = pl.pallas_call(
        _fused_kernel,
        grid=grid,
        in_specs=[
            pl.BlockSpec((BLOCK_B, D_IN), lambda i: (i, 0)),
            pl.BlockSpec((D_IN, D_FEAT), lambda i: (0, 0)),
            pl.BlockSpec((1, D_FEAT), lambda i: (0, 0)),
            pl.BlockSpec((D_FEAT, N_CLS), lambda i: (0, 0)),
            pl.BlockSpec((1, N_CLS), lambda i: (0, 0)),
        ],
        out_specs=[
            pl.BlockSpec((BLOCK_B, N_CLS), lambda i: (i, 0)),
            pl.BlockSpec((BLOCK_B, 1), lambda i: (i, 0)),
        ],
        out_shape=[
            jax.ShapeDtypeStruct((B, N_CLS), jnp.float32),
            jax.ShapeDtypeStruct((B, 1), jnp.int32),
        ],
    )(x, W_feat, b_feat.reshape(1, D_FEAT), W_fc, b_fc.reshape(1, N_CLS))
    return out, pred.reshape(B)


def kernel(x, W_feat, b_feat, W_fc, b_fc, epoch):
    del epoch
    return _run(x, W_feat, b_feat, W_fc, b_fc)


# speedup vs baseline: 1.3180x; 1.3180x over previous
import jax
import jax.numpy as jnp
from jax.experimental import pallas as pl

B, D_IN, D_FEAT, N_CLS = 16384, 128, 64, 5
BLOCK_B = 2048


def _fused_kernel(x_ref, wf_ref, bf_ref, wfc_ref, bfc_ref, bfc_col_ref,
                  out_ref, pred_ref):
    feats = jnp.maximum(
        jnp.dot(x_ref[...], wf_ref[...], preferred_element_type=jnp.float32)
        + bf_ref[...],
        0.0,
    )
    out = (
        jnp.dot(feats, wfc_ref[...], preferred_element_type=jnp.float32)
        + bfc_ref[...]
    )
    out_ref[...] = out
    out_t = (
        jax.lax.dot_general(
            wfc_ref[...], feats, (((0,), (1,)), ((), ())),
            preferred_element_type=jnp.float32,
        )
        + bfc_col_ref[...]
    )
    best = out_t[0:1, :]
    pred = jnp.zeros(best.shape, dtype=jnp.int32)
    for c in range(1, N_CLS):
        row = out_t[c:c + 1, :]
        better = row > best
        best = jnp.where(better, row, best)
        pred = jnp.where(better, c, pred)
    pred_ref[...] = pred


@jax.jit
def _run(x, W_feat, b_feat, W_fc, b_fc):
    grid = (B // BLOCK_B,)
    out, pred = pl.pallas_call(
        _fused_kernel,
        grid=grid,
        in_specs=[
            pl.BlockSpec((BLOCK_B, D_IN), lambda i: (i, 0)),
            pl.BlockSpec((D_IN, D_FEAT), lambda i: (0, 0)),
            pl.BlockSpec((1, D_FEAT), lambda i: (0, 0)),
            pl.BlockSpec((D_FEAT, N_CLS), lambda i: (0, 0)),
            pl.BlockSpec((1, N_CLS), lambda i: (0, 0)),
            pl.BlockSpec((N_CLS, 1), lambda i: (0, 0)),
        ],
        out_specs=[
            pl.BlockSpec((BLOCK_B, N_CLS), lambda i: (i, 0)),
            pl.BlockSpec((1, BLOCK_B), lambda i: (0, i)),
        ],
        out_shape=[
            jax.ShapeDtypeStruct((B, N_CLS), jnp.float32),
            jax.ShapeDtypeStruct((1, B), jnp.int32),
        ],
    )(
        x,
        W_feat,
        b_feat.reshape(1, D_FEAT),
        W_fc,
        b_fc.reshape(1, N_CLS),
        b_fc.reshape(N_CLS, 1),
    )
    return out, pred.reshape(B)


def kernel(x, W_feat, b_feat, W_fc, b_fc, epoch):
    del epoch
    return _run(x, W_feat, b_feat, W_fc, b_fc)
